# baseline (speedup 1.0000x reference)
import functools

import jax
import jax.numpy as jnp
from jax import lax
from jax.experimental import pallas as pl
from jax.experimental.pallas import tpu as pltpu
from jax.experimental.pallas import tpu_sc as plsc

VOCAB = 1000000
EMBED_DIM = 64
BATCH = 4096

_NC, _NS = 2, 16
_NW = _NC * _NS
_BPW = BATCH // _NW

@functools.cache
def _make_sc_gather():
    mesh = plsc.VectorSubcoreMesh(core_axis_name="c", subcore_axis_name="s")

    @functools.partial(
        pl.kernel,
        mesh=mesh,
        out_type=[
            jax.ShapeDtypeStruct((BATCH, EMBED_DIM), jnp.float32),
            jax.ShapeDtypeStruct((BATCH, EMBED_DIM), jnp.float32),
        ],
        scratch_types=[
            pltpu.VMEM((_BPW,), jnp.int32),
            pltpu.VMEM((_BPW, EMBED_DIM), jnp.float32),
            pltpu.VMEM((_BPW,), jnp.int32),
            pltpu.VMEM((_BPW, EMBED_DIM), jnp.float32),
            pltpu.SemaphoreType.DMA,
            pltpu.SemaphoreType.DMA,
        ],
        compiler_params=pltpu.CompilerParams(use_tc_tiling_on_sc=False),
    )
    def _sc_gather(cw_hbm, xw_hbm, ctab_hbm, xtab_hbm, outc_hbm, outx_hbm,
                   idx_c, rows_c, idx_x, rows_x, sem_c, sem_x):
        wid = lax.axis_index("s") * _NC + lax.axis_index("c")
        base = wid * _BPW
        pltpu.sync_copy(cw_hbm.at[pl.ds(base, _BPW)], idx_c)
        pltpu.sync_copy(xw_hbm.at[pl.ds(base, _BPW)], idx_x)
        cp_c = pltpu.async_copy(ctab_hbm.at[idx_c], rows_c, sem_c)
        cp_x = pltpu.async_copy(xtab_hbm.at[idx_x], rows_x, sem_x)
        cp_c.wait()
        cp_x.wait()
        pltpu.sync_copy(rows_c, outc_hbm.at[pl.ds(base, _BPW)])
        pltpu.sync_copy(rows_x, outx_hbm.at[pl.ds(base, _BPW)])

    return _sc_gather


_BM = 1024
_BN = 1024


def _mm_body(a_ref, b_ref, o_ref):
    o_ref[...] = lax.dot_general(
        a_ref[...], b_ref[...],
        (((1,), (1,)), ((), ())),
        preferred_element_type=jnp.float32,
    )


def _scores_matmul(center_embeds, context_embeds):
    return pl.pallas_call(
        _mm_body,
        grid=(BATCH // _BM, BATCH // _BN),
        in_specs=[
            pl.BlockSpec((_BM, EMBED_DIM), lambda i, j: (i, 0)),
            pl.BlockSpec((_BN, EMBED_DIM), lambda i, j: (j, 0)),
        ],
        out_specs=pl.BlockSpec((_BM, _BN), lambda i, j: (i, j)),
        out_shape=jax.ShapeDtypeStruct((BATCH, BATCH), jnp.float32),
    )(center_embeds, context_embeds)


def kernel(center_word, context_word, center_table, context_table):
    cw = center_word.astype(jnp.int32)
    xw = context_word.astype(jnp.int32)
    center_embeds, context_embeds = _make_sc_gather()(cw, xw, center_table,
                                                      context_table)
    return _scores_matmul(center_embeds, context_embeds)


# speedup vs baseline: 7.5142x; 7.5142x over previous
import functools

import jax
import jax.numpy as jnp
from jax import lax
from jax.experimental import pallas as pl
from jax.experimental.pallas import tpu as pltpu
from jax.experimental.pallas import tpu_sc as plsc

VOCAB = 1000000
EMBED_DIM = 64
BATCH = 4096

_NC, _NS = 2, 16
_NW = _NC * _NS
_BPW = BATCH // _NW
_RING = 8
_LANES = 128


def _gather_one_table(tab_hbm, idx_ref, colbuf, slabs, sem):
    vecs = [idx_ref[pl.ds(g * 16, 16)] for g in range(_BPW // 16)]
    rows_q = [jnp.arange(16, dtype=jnp.int32) + 16 * q
              for q in range(EMBED_DIM // 16)]
    handles = [None] * _BPW
    for i in range(_BPW + _RING):
        k = i - _RING
        if k >= 0:
            handles[k].wait()
            c = vecs[k // 16][k % 16] & (_LANES - 1)
            cols = jnp.full((16,), c, dtype=jnp.int32)
            slab = slabs[k % _RING]
            for q in range(EMBED_DIM // 16):
                vals = plsc.load_gather(slab, [rows_q[q], cols])
                colbuf[pl.ds(k * EMBED_DIM + 16 * q, 16)] = vals
        if i < _BPW:
            v = vecs[i // 16][i % 16]
            start = pl.multiple_of((v >> 7) << 7, _LANES)
            handles[i] = pltpu.async_copy(
                tab_hbm.at[:, pl.ds(start, _LANES)], slabs[i % _RING], sem)


@functools.cache
def _make_sc_gather():
    mesh = plsc.VectorSubcoreMesh(core_axis_name="c", subcore_axis_name="s")

    @functools.partial(
        pl.kernel,
        mesh=mesh,
        out_type=[
            jax.ShapeDtypeStruct((BATCH * EMBED_DIM,), jnp.float32),
            jax.ShapeDtypeStruct((BATCH * EMBED_DIM,), jnp.float32),
        ],
        scratch_types=[
            pltpu.VMEM((_BPW,), jnp.int32),
            pltpu.VMEM((_BPW,), jnp.int32),
            pltpu.VMEM((_BPW * EMBED_DIM,), jnp.float32),
            pltpu.VMEM((_BPW * EMBED_DIM,), jnp.float32),
        ] + [pltpu.VMEM((EMBED_DIM, _LANES), jnp.float32)
             for _ in range(_RING)] + [
            pltpu.SemaphoreType.DMA,
        ],
        compiler_params=pltpu.CompilerParams(use_tc_tiling_on_sc=True,
                                             needs_layout_passes=False),
    )
    def _sc_gather(cw_hbm, xw_hbm, ctabT_hbm, xtabT_hbm, outc_hbm, outx_hbm,
                   idx_c, idx_x, colbuf_c, colbuf_x, *rest):
        slabs = list(rest[:_RING])
        sem = rest[_RING]
        wid = lax.axis_index("s") * _NC + lax.axis_index("c")
        base = wid * _BPW
        pltpu.sync_copy(cw_hbm.at[pl.ds(base, _BPW)], idx_c)
        pltpu.sync_copy(xw_hbm.at[pl.ds(base, _BPW)], idx_x)
        _gather_one_table(ctabT_hbm, idx_c, colbuf_c, slabs, sem)
        _gather_one_table(xtabT_hbm, idx_x, colbuf_x, slabs, sem)
        pltpu.sync_copy(colbuf_c,
                        outc_hbm.at[pl.ds(base * EMBED_DIM, _BPW * EMBED_DIM)])
        pltpu.sync_copy(colbuf_x,
                        outx_hbm.at[pl.ds(base * EMBED_DIM, _BPW * EMBED_DIM)])

    return _sc_gather


_BM = 1024
_BN = 1024


def _mm_body(a_ref, b_ref, o_ref):
    o_ref[...] = lax.dot_general(
        a_ref[...], b_ref[...],
        (((1,), (1,)), ((), ())),
        preferred_element_type=jnp.float32,
    )


def _scores_matmul(center_embeds, context_embeds):
    return pl.pallas_call(
        _mm_body,
        grid=(BATCH // _BM, BATCH // _BN),
        in_specs=[
            pl.BlockSpec((_BM, EMBED_DIM), lambda i, j: (i, 0)),
            pl.BlockSpec((_BN, EMBED_DIM), lambda i, j: (j, 0)),
        ],
        out_specs=pl.BlockSpec((_BM, _BN), lambda i, j: (i, j)),
        out_shape=jax.ShapeDtypeStruct((BATCH, BATCH), jnp.float32),
    )(center_embeds, context_embeds)


def kernel(center_word, context_word, center_table, context_table):
    cw = center_word.astype(jnp.int32)
    xw = context_word.astype(jnp.int32)
    ctab_t = center_table.T
    xtab_t = context_table.T
    cflat, xflat = _make_sc_gather()(cw, xw, ctab_t, xtab_t)
    center_rows = cflat.reshape(BATCH, EMBED_DIM)
    context_rows = xflat.reshape(BATCH, EMBED_DIM)
    return _scores_matmul(center_rows, context_rows)


# speedup vs baseline: 7.6523x; 1.0184x over previous
import functools

import jax
import jax.numpy as jnp
from jax import lax
from jax.experimental import pallas as pl
from jax.experimental.pallas import tpu as pltpu
from jax.experimental.pallas import tpu_sc as plsc

VOCAB = 1000000
EMBED_DIM = 64
BATCH = 4096

_NC, _NS = 2, 16
_NW = _NC * _NS
_BPW = BATCH // _NW
_RING = 8
_LANES = 128


def _gather_one_table(tab_hbm, idx_ref, colbuf, slabs, sem):
    vecs = [idx_ref[pl.ds(g * 16, 16)] for g in range(_BPW // 16)]
    rows_q = [jnp.arange(16, dtype=jnp.int32) + 16 * q
              for q in range(EMBED_DIM // 16)]
    handles = [None] * _BPW
    for i in range(_BPW + _RING):
        k = i - _RING
        if k >= 0:
            handles[k].wait()
            c = vecs[k // 16][k % 16] & (_LANES - 1)
            cols = jnp.full((16,), c, dtype=jnp.int32)
            slab = slabs[k % _RING]
            for q in range(EMBED_DIM // 16):
                vals = plsc.load_gather(slab, [rows_q[q], cols])
                colbuf[pl.ds(k * EMBED_DIM + 16 * q, 16)] = vals
        if i < _BPW:
            v = vecs[i // 16][i % 16]
            start = pl.multiple_of((v >> 7) << 7, _LANES)
            handles[i] = pltpu.async_copy(
                tab_hbm.at[:, pl.ds(start, _LANES)], slabs[i % _RING], sem)


@functools.cache
def _make_sc_gather():
    mesh = plsc.VectorSubcoreMesh(core_axis_name="c", subcore_axis_name="s")

    @functools.partial(
        pl.kernel,
        mesh=mesh,
        out_type=[
            jax.ShapeDtypeStruct((BATCH * EMBED_DIM,), jnp.float32),
            jax.ShapeDtypeStruct((BATCH * EMBED_DIM,), jnp.float32),
        ],
        scratch_types=[
            pltpu.VMEM((_BPW,), jnp.int32),
            pltpu.VMEM((_BPW,), jnp.int32),
            pltpu.VMEM((_BPW * EMBED_DIM,), jnp.float32),
            pltpu.VMEM((_BPW * EMBED_DIM,), jnp.float32),
        ] + [pltpu.VMEM((EMBED_DIM, _LANES), jnp.float32)
             for _ in range(_RING)] + [
            pltpu.SemaphoreType.DMA,
        ],
        compiler_params=pltpu.CompilerParams(use_tc_tiling_on_sc=True,
                                             needs_layout_passes=False),
    )
    def _sc_gather(cw_hbm, xw_hbm, ctabT_hbm, xtabT_hbm, outc_hbm, outx_hbm,
                   idx_c, idx_x, colbuf_c, colbuf_x, *rest):
        slabs = list(rest[:_RING])
        sem = rest[_RING]
        wid = lax.axis_index("s") * _NC + lax.axis_index("c")
        base = wid * _BPW
        pltpu.sync_copy(cw_hbm.at[pl.ds(base, _BPW)], idx_c)
        pltpu.sync_copy(xw_hbm.at[pl.ds(base, _BPW)], idx_x)
        _gather_one_table(ctabT_hbm, idx_c, colbuf_c, slabs, sem)
        _gather_one_table(xtabT_hbm, idx_x, colbuf_x, slabs, sem)
        pltpu.sync_copy(colbuf_c,
                        outc_hbm.at[pl.ds(base * EMBED_DIM, _BPW * EMBED_DIM)])
        pltpu.sync_copy(colbuf_x,
                        outx_hbm.at[pl.ds(base * EMBED_DIM, _BPW * EMBED_DIM)])

    return _sc_gather


_BM = 512
_BN = 4096


def _mm_body(a_ref, b_ref, o_ref):
    o_ref[...] = lax.dot_general(
        a_ref[...], b_ref[...],
        (((1,), (1,)), ((), ())),
        preferred_element_type=jnp.float32,
    )


def _scores_matmul(center_embeds, context_embeds):
    return pl.pallas_call(
        _mm_body,
        grid=(BATCH // _BM, BATCH // _BN),
        in_specs=[
            pl.BlockSpec((_BM, EMBED_DIM), lambda i, j: (i, 0)),
            pl.BlockSpec((_BN, EMBED_DIM), lambda i, j: (j, 0)),
        ],
        out_specs=pl.BlockSpec((_BM, _BN), lambda i, j: (i, j)),
        out_shape=jax.ShapeDtypeStruct((BATCH, BATCH), jnp.float32),
    )(center_embeds, context_embeds)


def kernel(center_word, context_word, center_table, context_table):
    cw = center_word.astype(jnp.int32)
    xw = context_word.astype(jnp.int32)
    ctab_t = center_table.T
    xtab_t = context_table.T
    cflat, xflat = _make_sc_gather()(cw, xw, ctab_t, xtab_t)
    center_rows = cflat.reshape(BATCH, EMBED_DIM)
    context_rows = xflat.reshape(BATCH, EMBED_DIM)
    return _scores_matmul(center_rows, context_rows)
